# baseline (speedup 1.0000x reference)
import functools

import jax
import jax.numpy as jnp
from jax import lax
from jax.experimental import pallas as pl
from jax.experimental.pallas import tpu as pltpu
from jax.experimental.pallas import tpu_sc as plsc

_IT = False

TILE = 128
MTL = 16
NSLOTL = MTL * TILE
NF = 4
FT = 1024
F32 = jnp.float32
BF16 = jnp.bfloat16


def _ln(x, g, b, eps=1e-5):
    m = jnp.mean(x, axis=-1, keepdims=True)
    v = jnp.mean((x - m) ** 2, axis=-1, keepdims=True)
    return (x - m) / jnp.sqrt(v + eps) * g + b


def _route(logits):
    n = logits.shape[0]
    i8 = lax.broadcasted_iota(jnp.int32, (n, 8), 1)
    m1 = jnp.max(logits, axis=-1, keepdims=True)
    a1 = jnp.min(jnp.where(logits == m1, i8, 999), axis=-1, keepdims=True)
    oh1 = (i8 == a1).astype(F32)
    l2 = jnp.where(oh1 > 0, -jnp.inf, logits)
    m2 = jnp.max(l2, axis=-1, keepdims=True)
    a2 = jnp.min(jnp.where(l2 == m2, i8, 999), axis=-1, keepdims=True)
    oh2 = (i8 == a2).astype(F32)
    mask = oh1 + oh2
    e = jnp.exp(logits - m1) * mask
    gate = e / jnp.sum(e, axis=-1, keepdims=True)
    return oh1, oh2, gate


def _slotize(oha, ohb, gate, Lt, SU, iota16):
    dot = functools.partial(jnp.dot, preferred_element_type=F32)
    M = oha + ohb
    ranks = dot(Lt, M)
    counts = jnp.sum(M, axis=0, keepdims=True)
    pc = jnp.floor((counts + (TILE - 1)) / TILE) * TILE
    offs = dot(pc, SU)
    ends = offs + pc
    total = jnp.sum(pc, axis=-1, keepdims=True)
    posm = offs + ranks
    posA = jnp.sum(oha * posm, axis=-1, keepdims=True)
    posB = jnp.sum(ohb * posm, axis=-1, keepdims=True)
    gateA = jnp.sum(oha * gate, axis=-1, keepdims=True)
    gateB = jnp.sum(ohb * gate, axis=-1, keepdims=True)
    pk = posA + 2048.0 * posB
    gcat = jnp.concatenate([gateA, gateB], axis=0)
    sT = 128.0 * iota16
    raw = jnp.sum((sT >= ends).astype(F32), axis=-1, keepdims=True)
    glast = jnp.sum(((total - 128.0) >= ends).astype(F32), axis=-1,
                    keepdims=True)
    validT = sT < total
    tgrp = jnp.where(validT, raw, glast)
    txs = jnp.where(validT, iota16, total / 128.0 - 1.0)
    return (pk.astype(jnp.int32), gcat, tgrp.astype(jnp.int32),
            txs.astype(jnp.int32), validT.astype(jnp.int32))


def _stage_a_body(patches, Wp, bp, Wq, Wk, Wv, Wo, bo, pos, g1, b1, g2, b2,
                  g3, b3, Wg1, bg1, Wg2, bg2,
                  xn2_o, xn3_o, pk1_o, gc1_o, tg1_o, tx1_o, tv1_o,
                  pk2_o, gc2_o, tg2_o, tx2_o, tv2_o):
    dot = functools.partial(jnp.dot, preferred_element_type=F32)
    t = dot(patches[...], Wp[...]) + bp[...]
    xn1 = _ln(t, g1[...], b1[...])
    q = dot(xn1, Wq[...])
    k = dot(xn1, Wk[...])
    v = dot(xn1, Wv[...])
    S, hd = 64, 128
    scale = hd ** -0.5
    msk = (lax.broadcasted_iota(jnp.int32, (S, S), 0)
           >= lax.broadcasted_iota(jnp.int32, (S, S), 1))
    brows = []
    for bb in range(8):
        hcols = []
        for hh in range(8):
            qs = q[bb * S:(bb + 1) * S, hh * hd:(hh + 1) * hd]
            ks = k[bb * S:(bb + 1) * S, hh * hd:(hh + 1) * hd]
            vs = v[bb * S:(bb + 1) * S, hh * hd:(hh + 1) * hd]
            s = lax.dot_general(qs, ks, (((1,), (1,)), ((), ())),
                                preferred_element_type=F32) * scale
            s = jnp.where(msk, s, -jnp.inf)
            p = jnp.exp(s - jnp.max(s, axis=-1, keepdims=True))
            p = p / jnp.sum(p, axis=-1, keepdims=True)
            hcols.append(dot(p, vs))
        brows.append(jnp.concatenate(hcols, axis=1))
    ao = jnp.concatenate(brows, axis=0)
    t = t + dot(ao, Wo[...]) + bo[...]
    t = t + pos[...]
    xn2 = _ln(t, g2[...], b2[...])
    xn3 = _ln(t, g3[...], b3[...])
    lg1 = dot(xn2, Wg1[...]) + bg1[...]
    lg2 = dot(xn3, Wg2[...]) + bg2[...]

    Lt = (lax.broadcasted_iota(jnp.int32, (512, 512), 0)
          > lax.broadcasted_iota(jnp.int32, (512, 512), 1)).astype(F32)
    SU = (lax.broadcasted_iota(jnp.int32, (8, 8), 0)
          < lax.broadcasted_iota(jnp.int32, (8, 8), 1)).astype(F32)
    iota16 = lax.broadcasted_iota(jnp.int32, (MTL, 1), 0).astype(F32)

    oh1a, oh1b, gt1 = _route(lg1)
    pk1, gc1, tg1, tx1, tv1 = _slotize(oh1a, oh1b, gt1, Lt, SU, iota16)
    oh2a, oh2b, gt2 = _route(lg2)
    pk2, gc2, tg2, tx2, tv2 = _slotize(oh2a, oh2b, gt2, Lt, SU, iota16)

    xn2_o[...] = xn2.astype(BF16)
    xn3_o[...] = xn3.astype(BF16)
    pk1_o[...] = pk1; gc1_o[...] = gc1
    tg1_o[...] = tg1; tx1_o[...] = tx1; tv1_o[...] = tv1
    pk2_o[...] = pk2; gc2_o[...] = gc2
    tg2_o[...] = tg2; tx2_o[...] = tx2; tv2_o[...] = tv2


def _stage_a(patches, Wp, bp, Wq, Wk, Wv, Wo, bo, pos, g1, b1, g2, b2, g3, b3,
             Wg1, bg1, Wg2, bg2):
    i32c = lambda n: jax.ShapeDtypeStruct((n, 1), jnp.int32)
    f32c = lambda n: jax.ShapeDtypeStruct((n, 1), F32)
    outs = [
        jax.ShapeDtypeStruct((512, 1024), BF16),
        jax.ShapeDtypeStruct((512, 1024), BF16),
        i32c(512), f32c(1024), i32c(MTL), i32c(MTL), i32c(MTL),
        i32c(512), f32c(1024), i32c(MTL), i32c(MTL), i32c(MTL),
    ]
    return pl.pallas_call(_stage_a_body, out_shape=outs, interpret=_IT)(
        patches, Wp, bp, Wq, Wk, Wv, Wo, bo, pos, g1, b1, g2, b2, g3, b3,
        Wg1, bg1, Wg2, bg2)


_SC_MESH = plsc.VectorSubcoreMesh(core_axis_name="c", subcore_axis_name="s")
_SC_PARAMS = pltpu.CompilerParams(needs_layout_passes=False)


def _dispatch_body(xni, pk, gc, xsi, gsl,
                   pk_v, g_v, sidx_v, gsl_v, rows_v, sem, sem2):
    wid = lax.axis_index("s") * 2 + lax.axis_index("c")
    cpk = pltpu.async_copy(pk, pk_v, sem)
    cg = pltpu.async_copy(gc, g_v, sem2)
    cpk.wait()
    cg.wait()
    zi = jnp.zeros((16,), jnp.int32)
    zf = jnp.zeros((16,), F32)

    def zinit(i, carry):
        sidx_v[pl.ds(i * 16, 16)] = zi
        gsl_v[pl.ds(i * 16, 16)] = zf
        return carry
    lax.fori_loop(0, NSLOTL // 16, zinit, 0)

    def scat(i, carry):
        base = i * 16
        rvec = base + lax.iota(jnp.int32, 16)
        pkv = pk_v[pl.ds(base, 16)]
        pa = lax.bitwise_and(pkv, 2047)
        pb = lax.shift_right_logical(pkv, 11)
        plsc.store_scatter(sidx_v, [pa], rvec)
        plsc.store_scatter(sidx_v, [pb], rvec)
        plsc.store_scatter(gsl_v, [pa], g_v[pl.ds(base, 16)])
        plsc.store_scatter(gsl_v, [pb], g_v[pl.ds(base + 512, 16)])
        return carry
    lax.fori_loop(0, 512 // 16, scat, 0)

    c0 = wid * 64
    pltpu.async_copy(xni.at[sidx_v.at[pl.ds(c0, 64)]], rows_v, sem).wait()
    cw = pltpu.async_copy(rows_v, xsi.at[pl.ds(c0, 64)], sem)
    cgs = pltpu.async_copy(gsl_v.at[pl.ds(c0, 64)], gsl.at[pl.ds(c0, 64)],
                           sem2)
    cw.wait()
    cgs.wait()


def _dispatch_sc(xni, pk, gc):
    out_type = [
        jax.ShapeDtypeStruct((NSLOTL, 512), jnp.int32),
        jax.ShapeDtypeStruct((NSLOTL,), F32),
    ]
    scratch = [
        pltpu.VMEM((512,), jnp.int32),
        pltpu.VMEM((1024,), F32),
        pltpu.VMEM((NSLOTL,), jnp.int32),
        pltpu.VMEM((NSLOTL,), F32),
        pltpu.VMEM((64, 512), jnp.int32),
        pltpu.SemaphoreType.DMA,
        pltpu.SemaphoreType.DMA,
    ]
    fn = pl.kernel(_dispatch_body, out_type=out_type, mesh=_SC_MESH,
                   scratch_types=scratch, compiler_params=_SC_PARAMS)
    return fn(xni, pk, gc)


def _ffn_body(txs_s, tgrp_s, tval_s, xs_r, w1_r, b1_r, w2_r, b2_r, gsl_r,
              ys_r, acc_r):
    f = pl.program_id(1)

    @pl.when(tval_s[pl.program_id(0)] == 1)
    def _():
        xb = xs_r[...]
        h = jnp.maximum(
            jnp.dot(xb, w1_r[0], preferred_element_type=F32) + b1_r[0], 0.0)
        ctr = jnp.dot(h.astype(BF16), w2_r[0], preferred_element_type=F32)

        @pl.when(f == 0)
        def _():
            acc_r[...] = ctr + b2_r[0]

        @pl.when(f > 0)
        def _():
            acc_r[...] = acc_r[...] + ctr

        @pl.when(f == NF - 1)
        def _():
            ys_r[...] = acc_r[...] * gsl_r[...]


def _ffn_grouped(xs, gslot, txs, tgrp, tval, W1, b1, W2, b2):
    grid_spec = pltpu.PrefetchScalarGridSpec(
        num_scalar_prefetch=3,
        grid=(MTL, NF),
        in_specs=[
            pl.BlockSpec((TILE, 1024), lambda t, f, txs, tgrp, tval: (txs[t], 0)),
            pl.BlockSpec((1, 1024, FT), lambda t, f, txs, tgrp, tval: (tgrp[t], 0, f * tval[t])),
            pl.BlockSpec((1, 1, FT), lambda t, f, txs, tgrp, tval: (tgrp[t] * NF + f * tval[t], 0, 0)),
            pl.BlockSpec((1, FT, 1024), lambda t, f, txs, tgrp, tval: (tgrp[t], f * tval[t], 0)),
            pl.BlockSpec((1, 1, 1024), lambda t, f, txs, tgrp, tval: (tgrp[t], 0, 0)),
            pl.BlockSpec((TILE, 1), lambda t, f, txs, tgrp, tval: (txs[t], 0)),
        ],
        out_specs=pl.BlockSpec((TILE, 1024), lambda t, f, txs, tgrp, tval: (txs[t], 0)),
        scratch_shapes=[pltpu.VMEM((TILE, 1024), F32)],
    )
    return pl.pallas_call(
        _ffn_body,
        grid_spec=grid_spec,
        out_shape=jax.ShapeDtypeStruct((NSLOTL, 1024), F32),
        interpret=_IT,
    )(txs, tgrp, tval, xs, W1, b1.reshape(8 * NF, 1, FT), W2,
      b2.reshape(8, 1, 1024), gslot.reshape(NSLOTL, 1))


def _combine_body(ys, pk, out, pk_v, pa_v, pb_v, rA_v, rB_v, sem, sem2):
    wid = lax.axis_index("s") * 2 + lax.axis_index("c")
    r0 = wid * 16
    pltpu.sync_copy(pk.at[pl.ds(r0, 16)], pk_v)
    pkv = pk_v[...]
    pa_v[...] = lax.bitwise_and(pkv, 2047)
    pb_v[...] = lax.shift_right_logical(pkv, 11)
    ca = pltpu.async_copy(ys.at[pa_v], rA_v, sem)
    cb = pltpu.async_copy(ys.at[pb_v], rB_v, sem2)
    ca.wait()
    cb.wait()

    def addrow(j, carry):
        for kk in range(64):
            sl = pl.ds(kk * 16, 16)
            rA_v[j, sl] = rA_v[j, sl] + rB_v[j, sl]
        return carry
    lax.fori_loop(0, 16, addrow, 0)
    pltpu.sync_copy(rA_v, out.at[pl.ds(r0, 16)])


def _combine_sc(ys, pk):
    out_type = jax.ShapeDtypeStruct((512, 1024), F32)
    scratch = [
        pltpu.VMEM((16,), jnp.int32),
        pltpu.VMEM((16,), jnp.int32),
        pltpu.VMEM((16,), jnp.int32),
        pltpu.VMEM((16, 1024), F32),
        pltpu.VMEM((16, 1024), F32),
        pltpu.SemaphoreType.DMA,
        pltpu.SemaphoreType.DMA,
    ]
    fn = pl.kernel(_combine_body, out_type=out_type, mesh=_SC_MESH,
                   scratch_types=scratch, compiler_params=_SC_PARAMS)
    return fn(ys, pk)


def _head_body(sec_r, Wc_r, bc_r, feat_o, cls_o):
    rows = [jnp.mean(sec_r[bb * 64:(bb + 1) * 64, :], axis=0, keepdims=True)
            for bb in range(8)]
    feat = jnp.concatenate(rows, axis=0)
    feat_o[...] = feat
    cls_o[...] = jnp.dot(feat, Wc_r[...], preferred_element_type=F32) + bc_r[...]


def _head(second_rows, Wc, bc):
    outs = [jax.ShapeDtypeStruct((8, 1024), F32),
            jax.ShapeDtypeStruct((8, 10), F32)]
    return pl.pallas_call(_head_body, out_shape=outs, interpret=_IT)(
        second_rows, Wc, bc)


def _as_i32(xb):
    return lax.bitcast_convert_type(xb.reshape(xb.shape[0], 512, 2),
                                    jnp.int32)


def _as_bf16(xi):
    return lax.bitcast_convert_type(xi, BF16).reshape(xi.shape[0], 1024)


def kernel(x, W_patch, b_patch, Wq, Wk, Wv, Wo, bo, pos_emb, ln1_g, ln1_b,
           ln2_g, ln2_b, ln3_g, ln3_b, m1_Wg, m1_bg, m1_W1, m1_b1, m1_W2,
           m1_b2, m2_Wg, m2_bg, m2_W1, m2_b1, m2_W2, m2_b2, Wc, bc):
    b, c, h, w = x.shape
    P = 4
    hp, wp = h // P, w // P
    t = x.reshape(b, c, hp, P, wp, P).transpose(0, 1, 2, 4, 3, 5)
    t = t.reshape(b, c, hp * wp, P * P).transpose(0, 2, 1, 3)
    patches = t.reshape(b * hp * wp, c * P * P)
    pos512 = jnp.tile(pos_emb[0], (b, 1))
    row = lambda a: a.reshape(1, -1)

    (xn2b, xn3b, pk1, gc1, tg1, tx1, tv1,
     pk2, gc2, tg2, tx2, tv2) = _stage_a(
        patches, W_patch, row(b_patch), Wq, Wk, Wv, Wo, row(bo), pos512,
        row(ln1_g), row(ln1_b), row(ln2_g), row(ln2_b), row(ln3_g),
        row(ln3_b), m1_Wg, row(m1_bg), m2_Wg, row(m2_bg))

    pk1 = pk1.reshape(512)
    pk2 = pk2.reshape(512)
    xs1i, gsl1 = _dispatch_sc(_as_i32(xn2b), pk1, gc1.reshape(1024))
    xs2i, gsl2 = _dispatch_sc(_as_i32(xn3b), pk2, gc2.reshape(1024))

    ys1 = _ffn_grouped(_as_bf16(xs1i), gsl1, tx1.reshape(MTL),
                       tg1.reshape(MTL), tv1.reshape(MTL),
                       m1_W1.astype(BF16), m1_b1, m1_W2.astype(BF16), m1_b2)
    ys2 = _ffn_grouped(_as_bf16(xs2i), gsl2, tx2.reshape(MTL),
                       tg2.reshape(MTL), tv2.reshape(MTL),
                       m2_W1.astype(BF16), m2_b1, m2_W2.astype(BF16), m2_b2)

    out1 = _combine_sc(ys1, pk1)
    out2 = _combine_sc(ys2, pk2)
    first = out1.reshape(b, 64, 1024)
    second = out2.reshape(b, 64, 1024)
    feat, cls = _head(out2, Wc, row(bc))
    return first, second, feat, cls


# speedup vs baseline: 1.0757x; 1.0757x over previous
import functools

import jax
import jax.numpy as jnp
from jax import lax
from jax.experimental import pallas as pl
from jax.experimental.pallas import tpu as pltpu
from jax.experimental.pallas import tpu_sc as plsc

_IT = False

TILE = 128
MTL = 16
NSLOTL = MTL * TILE
NF = 4
FT = 1024
F32 = jnp.float32
BF16 = jnp.bfloat16


def _ln(x, g, b, eps=1e-5):
    m = jnp.mean(x, axis=-1, keepdims=True)
    v = jnp.mean((x - m) ** 2, axis=-1, keepdims=True)
    return (x - m) / jnp.sqrt(v + eps) * g + b


def _route(logits):
    n = logits.shape[0]
    i8 = lax.broadcasted_iota(jnp.int32, (n, 8), 1)
    m1 = jnp.max(logits, axis=-1, keepdims=True)
    a1 = jnp.min(jnp.where(logits == m1, i8, 999), axis=-1, keepdims=True)
    oh1 = (i8 == a1).astype(F32)
    l2 = jnp.where(oh1 > 0, -jnp.inf, logits)
    m2 = jnp.max(l2, axis=-1, keepdims=True)
    a2 = jnp.min(jnp.where(l2 == m2, i8, 999), axis=-1, keepdims=True)
    oh2 = (i8 == a2).astype(F32)
    mask = oh1 + oh2
    e = jnp.exp(logits - m1) * mask
    gate = e / jnp.sum(e, axis=-1, keepdims=True)
    return oh1, oh2, gate


def _slotize(oha, ohb, gate, Lt, SU, iota16):
    dot = functools.partial(jnp.dot, preferred_element_type=F32)
    M = oha + ohb
    ranks = dot(Lt, M)
    counts = jnp.sum(M, axis=0, keepdims=True)
    pc = jnp.floor((counts + (TILE - 1)) / TILE) * TILE
    offs = dot(pc, SU)
    ends = offs + pc
    total = jnp.sum(pc, axis=-1, keepdims=True)
    posm = offs + ranks
    posA = jnp.sum(oha * posm, axis=-1, keepdims=True)
    posB = jnp.sum(ohb * posm, axis=-1, keepdims=True)
    gateA = jnp.sum(oha * gate, axis=-1, keepdims=True)
    gateB = jnp.sum(ohb * gate, axis=-1, keepdims=True)
    pk = posA + 2048.0 * posB
    gcat = jnp.concatenate([gateA, gateB], axis=0)
    sT = 128.0 * iota16
    raw = jnp.sum((sT >= ends).astype(F32), axis=-1, keepdims=True)
    glast = jnp.sum(((total - 128.0) >= ends).astype(F32), axis=-1,
                    keepdims=True)
    validT = sT < total
    tgrp = jnp.where(validT, raw, glast)
    txs = jnp.where(validT, iota16, total / 128.0 - 1.0)
    return (pk.astype(jnp.int32), gcat, tgrp.astype(jnp.int32),
            txs.astype(jnp.int32), validT.astype(jnp.int32))


def _stage_a_body(patches, Wp, bp, Wq, Wk, Wv, Wo, bo, pos, g1, b1, g2, b2,
                  g3, b3, Wg1, bg1, Wg2, bg2,
                  xn2_o, xn3_o, pk1_o, gc1_o, tg1_o, tx1_o, tv1_o,
                  pk2_o, gc2_o, tg2_o, tx2_o, tv2_o):
    dot = functools.partial(jnp.dot, preferred_element_type=F32)
    t = dot(patches[...], Wp[...]) + bp[...]
    xn1 = _ln(t, g1[...], b1[...])
    q = dot(xn1, Wq[...])
    k = dot(xn1, Wk[...])
    v = dot(xn1, Wv[...])
    S, hd = 64, 128
    scale = hd ** -0.5
    msk = (lax.broadcasted_iota(jnp.int32, (S, S), 0)
           >= lax.broadcasted_iota(jnp.int32, (S, S), 1))
    brows = []
    for bb in range(8):
        hcols = []
        for hh in range(8):
            qs = q[bb * S:(bb + 1) * S, hh * hd:(hh + 1) * hd]
            ks = k[bb * S:(bb + 1) * S, hh * hd:(hh + 1) * hd]
            vs = v[bb * S:(bb + 1) * S, hh * hd:(hh + 1) * hd]
            s = lax.dot_general(qs, ks, (((1,), (1,)), ((), ())),
                                preferred_element_type=F32) * scale
            s = jnp.where(msk, s, -jnp.inf)
            p = jnp.exp(s - jnp.max(s, axis=-1, keepdims=True))
            p = p / jnp.sum(p, axis=-1, keepdims=True)
            hcols.append(dot(p, vs))
        brows.append(jnp.concatenate(hcols, axis=1))
    ao = jnp.concatenate(brows, axis=0)
    t = t + dot(ao, Wo[...]) + bo[...]
    t = t + pos[...]
    xn2 = _ln(t, g2[...], b2[...])
    xn3 = _ln(t, g3[...], b3[...])
    lg1 = dot(xn2, Wg1[...]) + bg1[...]
    lg2 = dot(xn3, Wg2[...]) + bg2[...]

    Lt = (lax.broadcasted_iota(jnp.int32, (512, 512), 0)
          > lax.broadcasted_iota(jnp.int32, (512, 512), 1)).astype(F32)
    SU = (lax.broadcasted_iota(jnp.int32, (8, 8), 0)
          < lax.broadcasted_iota(jnp.int32, (8, 8), 1)).astype(F32)
    iota16 = lax.broadcasted_iota(jnp.int32, (MTL, 1), 0).astype(F32)

    oh1a, oh1b, gt1 = _route(lg1)
    pk1, gc1, tg1, tx1, tv1 = _slotize(oh1a, oh1b, gt1, Lt, SU, iota16)
    oh2a, oh2b, gt2 = _route(lg2)
    pk2, gc2, tg2, tx2, tv2 = _slotize(oh2a, oh2b, gt2, Lt, SU, iota16)

    xn2_o[...] = xn2
    xn3_o[...] = xn3
    pk1_o[...] = pk1; gc1_o[...] = gc1
    tg1_o[...] = tg1; tx1_o[...] = tx1; tv1_o[...] = tv1
    pk2_o[...] = pk2; gc2_o[...] = gc2
    tg2_o[...] = tg2; tx2_o[...] = tx2; tv2_o[...] = tv2


def _stage_a(patches, Wp, bp, Wq, Wk, Wv, Wo, bo, pos, g1, b1, g2, b2, g3, b3,
             Wg1, bg1, Wg2, bg2):
    i32c = lambda n: jax.ShapeDtypeStruct((n, 1), jnp.int32)
    f32c = lambda n: jax.ShapeDtypeStruct((n, 1), F32)
    outs = [
        jax.ShapeDtypeStruct((512, 1024), F32),
        jax.ShapeDtypeStruct((512, 1024), F32),
        i32c(512), f32c(1024), i32c(MTL), i32c(MTL), i32c(MTL),
        i32c(512), f32c(1024), i32c(MTL), i32c(MTL), i32c(MTL),
    ]
    return pl.pallas_call(_stage_a_body, out_shape=outs, interpret=_IT)(
        patches, Wp, bp, Wq, Wk, Wv, Wo, bo, pos, g1, b1, g2, b2, g3, b3,
        Wg1, bg1, Wg2, bg2)


_SC_MESH = plsc.VectorSubcoreMesh(core_axis_name="c", subcore_axis_name="s")
_SC_PARAMS = pltpu.CompilerParams(needs_layout_passes=False)


def _dispatch_body(xn, pk, gc, xs, gsl,
                   pk_v, g_v, sidx_v, gsl_v, rows_v, sem, sem2):
    wid = lax.axis_index("s") * 2 + lax.axis_index("c")
    c0 = wid * 64
    cpk = pltpu.async_copy(pk, pk_v, sem)
    cg = pltpu.async_copy(gc, g_v, sem2)
    cpk.wait()
    cg.wait()
    zi = jnp.zeros((16,), jnp.int32)
    zf = jnp.zeros((16,), F32)
    for i in range(4):
        sidx_v[pl.ds(i * 16, 16)] = zi
        gsl_v[pl.ds(i * 16, 16)] = zf
    for i in range(32):
        base = i * 16
        rvec = base + lax.iota(jnp.int32, 16)
        pkv = pk_v[pl.ds(base, 16)]
        pa = lax.bitwise_and(pkv, 2047) - c0
        pb = lax.shift_right_logical(pkv, 11) - c0
        ma = (pa >= 0) & (pa < 64)
        mb = (pb >= 0) & (pb < 64)
        pa = lax.min(lax.max(pa, 0), 63)
        pb = lax.min(lax.max(pb, 0), 63)
        plsc.store_scatter(sidx_v, [pa], rvec, mask=ma)
        plsc.store_scatter(sidx_v, [pb], rvec, mask=mb)
        plsc.store_scatter(gsl_v, [pa], g_v[pl.ds(base, 16)], mask=ma)
        plsc.store_scatter(gsl_v, [pb], g_v[pl.ds(base + 512, 16)], mask=mb)
    pltpu.async_copy(xn.at[sidx_v], rows_v, sem).wait()
    cw = pltpu.async_copy(rows_v, xs.at[pl.ds(c0, 64)], sem)
    cgs = pltpu.async_copy(gsl_v, gsl.at[pl.ds(c0, 64)], sem2)
    cw.wait()
    cgs.wait()


def _dispatch_sc(xn, pk, gc):
    out_type = [
        jax.ShapeDtypeStruct((NSLOTL, 1024), F32),
        jax.ShapeDtypeStruct((NSLOTL,), F32),
    ]
    scratch = [
        pltpu.VMEM((512,), jnp.int32),
        pltpu.VMEM((1024,), F32),
        pltpu.VMEM((64,), jnp.int32),
        pltpu.VMEM((64,), F32),
        pltpu.VMEM((64, 1024), F32),
        pltpu.SemaphoreType.DMA,
        pltpu.SemaphoreType.DMA,
    ]
    fn = pl.kernel(_dispatch_body, out_type=out_type, mesh=_SC_MESH,
                   scratch_types=scratch, compiler_params=_SC_PARAMS)
    return fn(xn, pk, gc)


def _ffn_body(txs_s, tgrp_s, tval_s, xs_r, w1_r, b1_r, w2_r, b2_r, gsl_r,
              ys_r, acc_r):
    f = pl.program_id(1)

    @pl.when(tval_s[pl.program_id(0)] == 1)
    def _():
        xb = xs_r[...].astype(BF16)
        h = jnp.maximum(
            jnp.dot(xb, w1_r[0], preferred_element_type=F32) + b1_r[0], 0.0)
        ctr = jnp.dot(h.astype(BF16), w2_r[0], preferred_element_type=F32)

        @pl.when(f == 0)
        def _():
            acc_r[...] = ctr + b2_r[0]

        @pl.when(f > 0)
        def _():
            acc_r[...] = acc_r[...] + ctr

        @pl.when(f == NF - 1)
        def _():
            ys_r[...] = acc_r[...] * gsl_r[...]


def _ffn_grouped(xs, gslot, txs, tgrp, tval, W1, b1, W2, b2):
    grid_spec = pltpu.PrefetchScalarGridSpec(
        num_scalar_prefetch=3,
        grid=(MTL, NF),
        in_specs=[
            pl.BlockSpec((TILE, 1024), lambda t, f, txs, tgrp, tval: (txs[t], 0)),
            pl.BlockSpec((1, 1024, FT), lambda t, f, txs, tgrp, tval: (tgrp[t], 0, f * tval[t])),
            pl.BlockSpec((1, 1, FT), lambda t, f, txs, tgrp, tval: (tgrp[t] * NF + f * tval[t], 0, 0)),
            pl.BlockSpec((1, FT, 1024), lambda t, f, txs, tgrp, tval: (tgrp[t], f * tval[t], 0)),
            pl.BlockSpec((1, 1, 1024), lambda t, f, txs, tgrp, tval: (tgrp[t], 0, 0)),
            pl.BlockSpec((TILE, 1), lambda t, f, txs, tgrp, tval: (txs[t], 0)),
        ],
        out_specs=pl.BlockSpec((TILE, 1024), lambda t, f, txs, tgrp, tval: (txs[t], 0)),
        scratch_shapes=[pltpu.VMEM((TILE, 1024), F32)],
    )
    return pl.pallas_call(
        _ffn_body,
        grid_spec=grid_spec,
        out_shape=jax.ShapeDtypeStruct((NSLOTL, 1024), F32),
        interpret=_IT,
    )(txs, tgrp, tval, xs, W1, b1.reshape(8 * NF, 1, FT), W2,
      b2.reshape(8, 1, 1024), gslot.reshape(NSLOTL, 1))


def _combine_body(ys, pk, out, pk_v, pa_v, pb_v, rA_v, rB_v, sem, sem2):
    wid = lax.axis_index("s") * 2 + lax.axis_index("c")
    r0 = wid * 16
    pltpu.sync_copy(pk.at[pl.ds(r0, 16)], pk_v)
    pkv = pk_v[...]
    pa_v[...] = lax.bitwise_and(pkv, 2047)
    pb_v[...] = lax.shift_right_logical(pkv, 11)
    ca = pltpu.async_copy(ys.at[pa_v], rA_v, sem)
    cb = pltpu.async_copy(ys.at[pb_v], rB_v, sem2)
    ca.wait()
    cb.wait()

    def addrow(j, carry):
        for kk in range(64):
            sl = pl.ds(kk * 16, 16)
            rA_v[j, sl] = rA_v[j, sl] + rB_v[j, sl]
        return carry
    lax.fori_loop(0, 16, addrow, 0)
    pltpu.sync_copy(rA_v, out.at[pl.ds(r0, 16)])


def _combine_sc(ys, pk):
    out_type = jax.ShapeDtypeStruct((512, 1024), F32)
    scratch = [
        pltpu.VMEM((16,), jnp.int32),
        pltpu.VMEM((16,), jnp.int32),
        pltpu.VMEM((16,), jnp.int32),
        pltpu.VMEM((16, 1024), F32),
        pltpu.VMEM((16, 1024), F32),
        pltpu.SemaphoreType.DMA,
        pltpu.SemaphoreType.DMA,
    ]
    fn = pl.kernel(_combine_body, out_type=out_type, mesh=_SC_MESH,
                   scratch_types=scratch, compiler_params=_SC_PARAMS)
    return fn(ys, pk)


def _head_body(sec_r, Wc_r, bc_r, feat_o, cls_o):
    rows = [jnp.mean(sec_r[bb * 64:(bb + 1) * 64, :], axis=0, keepdims=True)
            for bb in range(8)]
    feat = jnp.concatenate(rows, axis=0)
    feat_o[...] = feat
    cls_o[...] = jnp.dot(feat, Wc_r[...], preferred_element_type=F32) + bc_r[...]


def _head(second_rows, Wc, bc):
    outs = [jax.ShapeDtypeStruct((8, 1024), F32),
            jax.ShapeDtypeStruct((8, 10), F32)]
    return pl.pallas_call(_head_body, out_shape=outs, interpret=_IT)(
        second_rows, Wc, bc)


def kernel(x, W_patch, b_patch, Wq, Wk, Wv, Wo, bo, pos_emb, ln1_g, ln1_b,
           ln2_g, ln2_b, ln3_g, ln3_b, m1_Wg, m1_bg, m1_W1, m1_b1, m1_W2,
           m1_b2, m2_Wg, m2_bg, m2_W1, m2_b1, m2_W2, m2_b2, Wc, bc):
    b, c, h, w = x.shape
    P = 4
    hp, wp = h // P, w // P
    t = x.reshape(b, c, hp, P, wp, P).transpose(0, 1, 2, 4, 3, 5)
    t = t.reshape(b, c, hp * wp, P * P).transpose(0, 2, 1, 3)
    patches = t.reshape(b * hp * wp, c * P * P)
    pos512 = jnp.tile(pos_emb[0], (b, 1))
    row = lambda a: a.reshape(1, -1)

    (xn2b, xn3b, pk1, gc1, tg1, tx1, tv1,
     pk2, gc2, tg2, tx2, tv2) = _stage_a(
        patches, W_patch, row(b_patch), Wq, Wk, Wv, Wo, row(bo), pos512,
        row(ln1_g), row(ln1_b), row(ln2_g), row(ln2_b), row(ln3_g),
        row(ln3_b), m1_Wg, row(m1_bg), m2_Wg, row(m2_bg))

    pk1 = pk1.reshape(512)
    pk2 = pk2.reshape(512)
    xs1, gsl1 = _dispatch_sc(xn2b, pk1, gc1.reshape(1024))
    xs2, gsl2 = _dispatch_sc(xn3b, pk2, gc2.reshape(1024))

    ys1 = _ffn_grouped(xs1, gsl1, tx1.reshape(MTL),
                       tg1.reshape(MTL), tv1.reshape(MTL),
                       m1_W1.astype(BF16), m1_b1, m1_W2.astype(BF16), m1_b2)
    ys2 = _ffn_grouped(xs2, gsl2, tx2.reshape(MTL),
                       tg2.reshape(MTL), tv2.reshape(MTL),
                       m2_W1.astype(BF16), m2_b1, m2_W2.astype(BF16), m2_b2)

    out1 = _combine_sc(ys1, pk1)
    out2 = _combine_sc(ys2, pk2)
    first = out1.reshape(b, 64, 1024)
    second = out2.reshape(b, 64, 1024)
    feat, cls = _head(out2, Wc, row(bc))
    return first, second, feat, cls


# speedup vs baseline: 1.9812x; 1.8419x over previous
import functools

import jax
import jax.numpy as jnp
from jax import lax
from jax.experimental import pallas as pl
from jax.experimental.pallas import tpu as pltpu
from jax.experimental.pallas import tpu_sc as plsc

_IT = False

TILE = 128
MTL = 16
NSLOTL = MTL * TILE
NF = 4
FT = 1024
F32 = jnp.float32
BF16 = jnp.bfloat16


def _ln(x, g, b, eps=1e-5):
    m = jnp.mean(x, axis=-1, keepdims=True)
    v = jnp.mean((x - m) ** 2, axis=-1, keepdims=True)
    return (x - m) / jnp.sqrt(v + eps) * g + b


def _route(logits):
    n = logits.shape[0]
    i8 = lax.broadcasted_iota(jnp.int32, (n, 8), 1)
    m1 = jnp.max(logits, axis=-1, keepdims=True)
    a1 = jnp.min(jnp.where(logits == m1, i8, 999), axis=-1, keepdims=True)
    oh1 = (i8 == a1).astype(F32)
    l2 = jnp.where(oh1 > 0, -jnp.inf, logits)
    m2 = jnp.max(l2, axis=-1, keepdims=True)
    a2 = jnp.min(jnp.where(l2 == m2, i8, 999), axis=-1, keepdims=True)
    oh2 = (i8 == a2).astype(F32)
    mask = oh1 + oh2
    e = jnp.exp(logits - m1) * mask
    gate = e / jnp.sum(e, axis=-1, keepdims=True)
    return oh1, oh2, gate


def _slotize(oha, ohb, gate, Lt, SU, iota16):
    dot = functools.partial(jnp.dot, preferred_element_type=F32)
    M = oha + ohb
    ranks = dot(Lt, M)
    counts = jnp.sum(M, axis=0, keepdims=True)
    pc = jnp.floor((counts + (TILE - 1)) / TILE) * TILE
    offs = dot(pc, SU)
    ends = offs + pc
    total = jnp.sum(pc, axis=-1, keepdims=True)
    posm = offs + ranks
    posA = jnp.sum(oha * posm, axis=-1, keepdims=True)
    posB = jnp.sum(ohb * posm, axis=-1, keepdims=True)
    gateA = jnp.sum(oha * gate, axis=-1, keepdims=True)
    gateB = jnp.sum(ohb * gate, axis=-1, keepdims=True)
    pk = posA + 2048.0 * posB
    gcat = jnp.concatenate([gateA, gateB], axis=0)
    sT = 128.0 * iota16
    raw = jnp.sum((sT >= ends).astype(F32), axis=-1, keepdims=True)
    glast = jnp.sum(((total - 128.0) >= ends).astype(F32), axis=-1,
                    keepdims=True)
    validT = sT < total
    tgrp = jnp.where(validT, raw, glast)
    txs = jnp.where(validT, iota16, total / 128.0 - 1.0)
    return (pk.astype(jnp.int32), gcat, tgrp.astype(jnp.int32),
            txs.astype(jnp.int32), validT.astype(jnp.int32))


def _stage_a_body(patches, Wp, bp, Wq, Wk, Wv, Wo, bo, pos, g1, b1, g2, b2,
                  g3, b3, Wg1, bg1, Wg2, bg2,
                  xn2_o, xn3_o, pk1_o, gc1_o, tg1_o, tx1_o, tv1_o,
                  pk2_o, gc2_o, tg2_o, tx2_o, tv2_o):
    dot = functools.partial(jnp.dot, preferred_element_type=F32)
    t = dot(patches[...], Wp[...]) + bp[...]
    xn1 = _ln(t, g1[...], b1[...])
    q = dot(xn1, Wq[...])
    k = dot(xn1, Wk[...])
    v = dot(xn1, Wv[...])
    S, hd = 64, 128
    scale = hd ** -0.5
    msk = (lax.broadcasted_iota(jnp.int32, (S, S), 0)
           >= lax.broadcasted_iota(jnp.int32, (S, S), 1))
    brows = []
    for bb in range(8):
        hcols = []
        for hh in range(8):
            qs = q[bb * S:(bb + 1) * S, hh * hd:(hh + 1) * hd]
            ks = k[bb * S:(bb + 1) * S, hh * hd:(hh + 1) * hd]
            vs = v[bb * S:(bb + 1) * S, hh * hd:(hh + 1) * hd]
            s = lax.dot_general(qs, ks, (((1,), (1,)), ((), ())),
                                preferred_element_type=F32) * scale
            s = jnp.where(msk, s, -jnp.inf)
            p = jnp.exp(s - jnp.max(s, axis=-1, keepdims=True))
            p = p / jnp.sum(p, axis=-1, keepdims=True)
            hcols.append(dot(p, vs))
        brows.append(jnp.concatenate(hcols, axis=1))
    ao = jnp.concatenate(brows, axis=0)
    t = t + dot(ao, Wo[...]) + bo[...]
    t = t + pos[...]
    xn2 = _ln(t, g2[...], b2[...])
    xn3 = _ln(t, g3[...], b3[...])
    lg1 = dot(xn2, Wg1[...]) + bg1[...]
    lg2 = dot(xn3, Wg2[...]) + bg2[...]

    Lt = (lax.broadcasted_iota(jnp.int32, (512, 512), 0)
          > lax.broadcasted_iota(jnp.int32, (512, 512), 1)).astype(F32)
    SU = (lax.broadcasted_iota(jnp.int32, (8, 8), 0)
          < lax.broadcasted_iota(jnp.int32, (8, 8), 1)).astype(F32)
    iota16 = lax.broadcasted_iota(jnp.int32, (MTL, 1), 0).astype(F32)

    oh1a, oh1b, gt1 = _route(lg1)
    pk1, gc1, tg1, tx1, tv1 = _slotize(oh1a, oh1b, gt1, Lt, SU, iota16)
    oh2a, oh2b, gt2 = _route(lg2)
    pk2, gc2, tg2, tx2, tv2 = _slotize(oh2a, oh2b, gt2, Lt, SU, iota16)

    xn2_o[...] = xn2
    xn3_o[...] = xn3
    pk1_o[...] = pk1; gc1_o[...] = gc1
    tg1_o[...] = tg1; tx1_o[...] = tx1; tv1_o[...] = tv1
    pk2_o[...] = pk2; gc2_o[...] = gc2
    tg2_o[...] = tg2; tx2_o[...] = tx2; tv2_o[...] = tv2


def _stage_a(patches, Wp, bp, Wq, Wk, Wv, Wo, bo, pos, g1, b1, g2, b2, g3, b3,
             Wg1, bg1, Wg2, bg2):
    i32c = lambda n: jax.ShapeDtypeStruct((n, 1), jnp.int32)
    f32c = lambda n: jax.ShapeDtypeStruct((n, 1), F32)
    outs = [
        jax.ShapeDtypeStruct((512, 1024), F32),
        jax.ShapeDtypeStruct((512, 1024), F32),
        i32c(512), f32c(1024), i32c(MTL), i32c(MTL), i32c(MTL),
        i32c(512), f32c(1024), i32c(MTL), i32c(MTL), i32c(MTL),
    ]
    return pl.pallas_call(_stage_a_body, out_shape=outs, interpret=_IT)(
        patches, Wp, bp, Wq, Wk, Wv, Wo, bo, pos, g1, b1, g2, b2, g3, b3,
        Wg1, bg1, Wg2, bg2)


_SC_MESH = plsc.VectorSubcoreMesh(core_axis_name="c", subcore_axis_name="s")
_SC_PARAMS = pltpu.CompilerParams(needs_layout_passes=False)


def _ffn_body(txs_s, tgrp_s, tval_s, pa_r, pb_r, ga_r, gb_r, xn_r,
              w1_r, b1_r, w2_r, b2_r, ys_r, xg_r, gs_r, acc_r):
    t = pl.program_id(0)
    f = pl.program_id(1)

    @pl.when(tval_s[t] == 1)
    def _():
        @pl.when(f == 0)
        def _():
            si = t * TILE + lax.broadcasted_iota(jnp.int32, (TILE, 1), 0)
            ohA = (pa_r[...] == si).astype(F32)
            ohB = (pb_r[...] == si).astype(F32)
            xg_r[...] = jnp.dot(ohA + ohB, xn_r[...],
                                preferred_element_type=F32)
            gs_r[...] = (jnp.dot(ohA, ga_r[...], preferred_element_type=F32)
                         + jnp.dot(ohB, gb_r[...],
                                   preferred_element_type=F32))

        xb = xg_r[...]
        h = jnp.maximum(
            jnp.dot(xb, w1_r[0], preferred_element_type=F32) + b1_r[0], 0.0)
        ctr = jnp.dot(h, w2_r[0], preferred_element_type=F32)

        @pl.when(f == 0)
        def _():
            acc_r[...] = ctr + b2_r[0]

        @pl.when(f > 0)
        def _():
            acc_r[...] = acc_r[...] + ctr

        @pl.when(f == NF - 1)
        def _():
            ys_r[...] = acc_r[...] * gs_r[...]


def _ffn_grouped(xn, paR, pbR, gaC, gbC, txs, tgrp, tval, W1, b1, W2, b2):
    grid_spec = pltpu.PrefetchScalarGridSpec(
        num_scalar_prefetch=3,
        grid=(MTL, NF),
        in_specs=[
            pl.BlockSpec((1, 512), lambda t, f, txs, tgrp, tval: (0, 0)),
            pl.BlockSpec((1, 512), lambda t, f, txs, tgrp, tval: (0, 0)),
            pl.BlockSpec((512, 1), lambda t, f, txs, tgrp, tval: (0, 0)),
            pl.BlockSpec((512, 1), lambda t, f, txs, tgrp, tval: (0, 0)),
            pl.BlockSpec((512, 1024), lambda t, f, txs, tgrp, tval: (0, 0)),
            pl.BlockSpec((1, 1024, FT), lambda t, f, txs, tgrp, tval: (tgrp[t], 0, f * tval[t])),
            pl.BlockSpec((1, 1, FT), lambda t, f, txs, tgrp, tval: (tgrp[t] * NF + f * tval[t], 0, 0)),
            pl.BlockSpec((1, FT, 1024), lambda t, f, txs, tgrp, tval: (tgrp[t], f * tval[t], 0)),
            pl.BlockSpec((1, 1, 1024), lambda t, f, txs, tgrp, tval: (tgrp[t], 0, 0)),
        ],
        out_specs=pl.BlockSpec((TILE, 1024), lambda t, f, txs, tgrp, tval: (txs[t], 0)),
        scratch_shapes=[pltpu.VMEM((TILE, 1024), F32),
                        pltpu.VMEM((TILE, 1), F32),
                        pltpu.VMEM((TILE, 1024), F32)],
    )
    return pl.pallas_call(
        _ffn_body,
        grid_spec=grid_spec,
        out_shape=jax.ShapeDtypeStruct((NSLOTL, 1024), F32),
        interpret=_IT,
    )(txs, tgrp, tval, paR, pbR, gaC, gbC, xn, W1,
      b1.reshape(8 * NF, 1, FT), W2, b2.reshape(8, 1, 1024))


def _combine_body(ys, pk, out, pk_v, pa_v, pb_v, rA_v, rB_v, sem, sem2):
    wid = lax.axis_index("s") * 2 + lax.axis_index("c")
    r0 = wid * 16
    pltpu.sync_copy(pk.at[pl.ds(r0, 16)], pk_v)
    pkv = pk_v[...]
    pa_v[...] = lax.bitwise_and(pkv, 2047)
    pb_v[...] = lax.shift_right_logical(pkv, 11)
    ca = pltpu.async_copy(ys.at[pa_v], rA_v, sem)
    cb = pltpu.async_copy(ys.at[pb_v], rB_v, sem2)
    ca.wait()
    cb.wait()

    def addrow(j, carry):
        for kk in range(64):
            sl = pl.ds(kk * 16, 16)
            rA_v[j, sl] = rA_v[j, sl] + rB_v[j, sl]
        return carry
    lax.fori_loop(0, 16, addrow, 0)
    pltpu.sync_copy(rA_v, out.at[pl.ds(r0, 16)])


def _combine_sc(ys, pk):
    out_type = jax.ShapeDtypeStruct((512, 1024), F32)
    scratch = [
        pltpu.VMEM((16,), jnp.int32),
        pltpu.VMEM((16,), jnp.int32),
        pltpu.VMEM((16,), jnp.int32),
        pltpu.VMEM((16, 1024), F32),
        pltpu.VMEM((16, 1024), F32),
        pltpu.SemaphoreType.DMA,
        pltpu.SemaphoreType.DMA,
    ]
    fn = pl.kernel(_combine_body, out_type=out_type, mesh=_SC_MESH,
                   scratch_types=scratch, compiler_params=_SC_PARAMS)
    return fn(ys, pk)


def _head_body(sec_r, Wc_r, bc_r, feat_o, cls_o):
    rows = [jnp.mean(sec_r[bb * 64:(bb + 1) * 64, :], axis=0, keepdims=True)
            for bb in range(8)]
    feat = jnp.concatenate(rows, axis=0)
    feat_o[...] = feat
    cls_o[...] = jnp.dot(feat, Wc_r[...], preferred_element_type=F32) + bc_r[...]


def _head(second_rows, Wc, bc):
    outs = [jax.ShapeDtypeStruct((8, 1024), F32),
            jax.ShapeDtypeStruct((8, 10), F32)]
    return pl.pallas_call(_head_body, out_shape=outs, interpret=_IT)(
        second_rows, Wc, bc)


def kernel(x, W_patch, b_patch, Wq, Wk, Wv, Wo, bo, pos_emb, ln1_g, ln1_b,
           ln2_g, ln2_b, ln3_g, ln3_b, m1_Wg, m1_bg, m1_W1, m1_b1, m1_W2,
           m1_b2, m2_Wg, m2_bg, m2_W1, m2_b1, m2_W2, m2_b2, Wc, bc):
    b, c, h, w = x.shape
    P = 4
    hp, wp = h // P, w // P
    t = x.reshape(b, c, hp, P, wp, P).transpose(0, 1, 2, 4, 3, 5)
    t = t.reshape(b, c, hp * wp, P * P).transpose(0, 2, 1, 3)
    patches = t.reshape(b * hp * wp, c * P * P)
    pos512 = jnp.tile(pos_emb[0], (b, 1))
    row = lambda a: a.reshape(1, -1)

    (xn2b, xn3b, pk1, gc1, tg1, tx1, tv1,
     pk2, gc2, tg2, tx2, tv2) = _stage_a(
        patches, W_patch, row(b_patch), Wq, Wk, Wv, Wo, row(bo), pos512,
        row(ln1_g), row(ln1_b), row(ln2_g), row(ln2_b), row(ln3_g),
        row(ln3_b), m1_Wg, row(m1_bg), m2_Wg, row(m2_bg))

    pk1 = pk1.reshape(512)
    pk2 = pk2.reshape(512)
    pa1 = (pk1 & 2047).reshape(1, 512)
    pb1 = (pk1 >> 11).reshape(1, 512)
    pa2 = (pk2 & 2047).reshape(1, 512)
    pb2 = (pk2 >> 11).reshape(1, 512)
    ga1, gb1 = gc1[:512], gc1[512:]
    ga2, gb2 = gc2[:512], gc2[512:]

    ys1 = _ffn_grouped(xn2b, pa1, pb1, ga1, gb1, tx1.reshape(MTL),
                       tg1.reshape(MTL), tv1.reshape(MTL),
                       m1_W1, m1_b1, m1_W2, m1_b2)
    ys2 = _ffn_grouped(xn3b, pa2, pb2, ga2, gb2, tx2.reshape(MTL),
                       tg2.reshape(MTL), tv2.reshape(MTL),
                       m2_W1, m2_b1, m2_W2, m2_b2)

    out1 = _combine_sc(ys1, pk1)
    out2 = _combine_sc(ys2, pk2)
    first = out1.reshape(b, 64, 1024)
    second = out2.reshape(b, 64, 1024)
    feat, cls = _head(out2, Wc, row(bc))
    return first, second, feat, cls


# speedup vs baseline: 1.9871x; 1.0030x over previous
import functools

import jax
import jax.numpy as jnp
from jax import lax
from jax.experimental import pallas as pl
from jax.experimental.pallas import tpu as pltpu
from jax.experimental.pallas import tpu_sc as plsc

_IT = False

TILE = 128
MTL = 16
NSLOTL = MTL * TILE
NF = 4
FT = 1024
F32 = jnp.float32
BF16 = jnp.bfloat16


def _ln(x, g, b, eps=1e-5):
    m = jnp.mean(x, axis=-1, keepdims=True)
    v = jnp.mean((x - m) ** 2, axis=-1, keepdims=True)
    return (x - m) / jnp.sqrt(v + eps) * g + b


def _route(logits):
    n = logits.shape[0]
    i8 = lax.broadcasted_iota(jnp.int32, (n, 8), 1)
    m1 = jnp.max(logits, axis=-1, keepdims=True)
    a1 = jnp.min(jnp.where(logits == m1, i8, 999), axis=-1, keepdims=True)
    oh1 = (i8 == a1).astype(F32)
    l2 = jnp.where(oh1 > 0, -jnp.inf, logits)
    m2 = jnp.max(l2, axis=-1, keepdims=True)
    a2 = jnp.min(jnp.where(l2 == m2, i8, 999), axis=-1, keepdims=True)
    oh2 = (i8 == a2).astype(F32)
    mask = oh1 + oh2
    e = jnp.exp(logits - m1) * mask
    gate = e / jnp.sum(e, axis=-1, keepdims=True)
    return oh1, oh2, gate


def _slotize(oha, ohb, gate, Lt, SU, iota16):
    dot = functools.partial(jnp.dot, preferred_element_type=F32)
    M = oha + ohb
    ranks = dot(Lt, M)
    counts = jnp.sum(M, axis=0, keepdims=True)
    pc = jnp.floor((counts + (TILE - 1)) / TILE) * TILE
    offs = dot(pc, SU)
    ends = offs + pc
    total = jnp.sum(pc, axis=-1, keepdims=True)
    posm = offs + ranks
    posA = jnp.sum(oha * posm, axis=-1, keepdims=True)
    posB = jnp.sum(ohb * posm, axis=-1, keepdims=True)
    gateA = jnp.sum(oha * gate, axis=-1, keepdims=True)
    gateB = jnp.sum(ohb * gate, axis=-1, keepdims=True)
    pk = posA + 2048.0 * posB
    gcat = jnp.concatenate([gateA, gateB], axis=0)
    sT = 128.0 * iota16
    raw = jnp.sum((sT >= ends).astype(F32), axis=-1, keepdims=True)
    glast = jnp.sum(((total - 128.0) >= ends).astype(F32), axis=-1,
                    keepdims=True)
    validT = sT < total
    tgrp = jnp.where(validT, raw, glast)
    txs = jnp.where(validT, iota16, total / 128.0 - 1.0)
    return (pk.astype(jnp.int32), gcat, tgrp.astype(jnp.int32),
            txs.astype(jnp.int32), validT.astype(jnp.int32))


def _stage_a_body(patches, Wp, bp, Wq, Wk, Wv, Wo, bo, pos, g1, b1, g2, b2,
                  g3, b3, Wg1, bg1, Wg2, bg2,
                  xn2_o, xn3_o, pk1_o, gc1_o, tg1_o, tx1_o, tv1_o,
                  pk2_o, gc2_o, tg2_o, tx2_o, tv2_o):
    dot = functools.partial(jnp.dot, preferred_element_type=F32)
    t = dot(patches[...], Wp[...]) + bp[...]
    xn1 = _ln(t, g1[...], b1[...])
    q = dot(xn1, Wq[...])
    k = dot(xn1, Wk[...])
    v = dot(xn1, Wv[...])
    S, hd = 64, 128
    scale = hd ** -0.5
    msk = (lax.broadcasted_iota(jnp.int32, (S, S), 0)
           >= lax.broadcasted_iota(jnp.int32, (S, S), 1))
    brows = []
    for bb in range(8):
        hcols = []
        for hh in range(8):
            qs = q[bb * S:(bb + 1) * S, hh * hd:(hh + 1) * hd]
            ks = k[bb * S:(bb + 1) * S, hh * hd:(hh + 1) * hd]
            vs = v[bb * S:(bb + 1) * S, hh * hd:(hh + 1) * hd]
            s = lax.dot_general(qs, ks, (((1,), (1,)), ((), ())),
                                preferred_element_type=F32) * scale
            s = jnp.where(msk, s, -jnp.inf)
            p = jnp.exp(s - jnp.max(s, axis=-1, keepdims=True))
            p = p / jnp.sum(p, axis=-1, keepdims=True)
            hcols.append(dot(p, vs))
        brows.append(jnp.concatenate(hcols, axis=1))
    ao = jnp.concatenate(brows, axis=0)
    t = t + dot(ao, Wo[...]) + bo[...]
    t = t + pos[...]
    xn2 = _ln(t, g2[...], b2[...])
    xn3 = _ln(t, g3[...], b3[...])
    lg1 = dot(xn2, Wg1[...]) + bg1[...]
    lg2 = dot(xn3, Wg2[...]) + bg2[...]

    Lt = (lax.broadcasted_iota(jnp.int32, (512, 512), 0)
          > lax.broadcasted_iota(jnp.int32, (512, 512), 1)).astype(F32)
    SU = (lax.broadcasted_iota(jnp.int32, (8, 8), 0)
          < lax.broadcasted_iota(jnp.int32, (8, 8), 1)).astype(F32)
    iota16 = lax.broadcasted_iota(jnp.int32, (MTL, 1), 0).astype(F32)

    oh1a, oh1b, gt1 = _route(lg1)
    pk1, gc1, tg1, tx1, tv1 = _slotize(oh1a, oh1b, gt1, Lt, SU, iota16)
    oh2a, oh2b, gt2 = _route(lg2)
    pk2, gc2, tg2, tx2, tv2 = _slotize(oh2a, oh2b, gt2, Lt, SU, iota16)

    xn2_o[...] = xn2
    xn3_o[...] = xn3
    pk1_o[...] = pk1; gc1_o[...] = gc1
    tg1_o[...] = tg1; tx1_o[...] = tx1; tv1_o[...] = tv1
    pk2_o[...] = pk2; gc2_o[...] = gc2
    tg2_o[...] = tg2; tx2_o[...] = tx2; tv2_o[...] = tv2


def _stage_a(patches, Wp, bp, Wq, Wk, Wv, Wo, bo, pos, g1, b1, g2, b2, g3, b3,
             Wg1, bg1, Wg2, bg2):
    i32c = lambda n: jax.ShapeDtypeStruct((n, 1), jnp.int32)
    f32c = lambda n: jax.ShapeDtypeStruct((n, 1), F32)
    outs = [
        jax.ShapeDtypeStruct((512, 1024), F32),
        jax.ShapeDtypeStruct((512, 1024), F32),
        i32c(512), f32c(1024), i32c(MTL), i32c(MTL), i32c(MTL),
        i32c(512), f32c(1024), i32c(MTL), i32c(MTL), i32c(MTL),
    ]
    return pl.pallas_call(_stage_a_body, out_shape=outs, interpret=_IT)(
        patches, Wp, bp, Wq, Wk, Wv, Wo, bo, pos, g1, b1, g2, b2, g3, b3,
        Wg1, bg1, Wg2, bg2)


_SC_MESH = plsc.VectorSubcoreMesh(core_axis_name="c", subcore_axis_name="s")
_SC_PARAMS = pltpu.CompilerParams(needs_layout_passes=False)


def _ffn_body(txs_s, tgrp_s, tval_s, pa_r, pb_r, ga_r, gb_r, xn_r,
              w1_r, b1_r, w2_r, b2_r, ys_r, xg_r, gs_r, acc_r):
    t = pl.program_id(0)
    f = pl.program_id(1)

    @pl.when(tval_s[t] == 1)
    def _():
        @pl.when(f == 0)
        def _():
            si = t * TILE + lax.broadcasted_iota(jnp.int32, (TILE, 1), 0)
            ohA = (pa_r[...] == si).astype(F32)
            ohB = (pb_r[...] == si).astype(F32)
            xg_r[...] = jnp.dot(ohA + ohB, xn_r[...],
                                preferred_element_type=F32)
            gs_r[...] = (jnp.dot(ohA, ga_r[...], preferred_element_type=F32)
                         + jnp.dot(ohB, gb_r[...],
                                   preferred_element_type=F32))

        xb = xg_r[...].astype(BF16)
        h = jnp.maximum(
            jnp.dot(xb, w1_r[0].astype(BF16),
                    preferred_element_type=F32) + b1_r[0], 0.0)
        ctr = jnp.dot(h.astype(BF16), w2_r[0].astype(BF16),
                      preferred_element_type=F32)

        @pl.when(f == 0)
        def _():
            acc_r[...] = ctr + b2_r[0]

        @pl.when(f > 0)
        def _():
            acc_r[...] = acc_r[...] + ctr

        @pl.when(f == NF - 1)
        def _():
            ys_r[...] = acc_r[...] * gs_r[...]


def _ffn_grouped(xn, paR, pbR, gaC, gbC, txs, tgrp, tval, W1, b1, W2, b2):
    grid_spec = pltpu.PrefetchScalarGridSpec(
        num_scalar_prefetch=3,
        grid=(MTL, NF),
        in_specs=[
            pl.BlockSpec((1, 512), lambda t, f, txs, tgrp, tval: (0, 0)),
            pl.BlockSpec((1, 512), lambda t, f, txs, tgrp, tval: (0, 0)),
            pl.BlockSpec((512, 1), lambda t, f, txs, tgrp, tval: (0, 0)),
            pl.BlockSpec((512, 1), lambda t, f, txs, tgrp, tval: (0, 0)),
            pl.BlockSpec((512, 1024), lambda t, f, txs, tgrp, tval: (0, 0)),
            pl.BlockSpec((1, 1024, FT), lambda t, f, txs, tgrp, tval: (tgrp[t], 0, f * tval[t])),
            pl.BlockSpec((1, 1, FT), lambda t, f, txs, tgrp, tval: (tgrp[t] * NF + f * tval[t], 0, 0)),
            pl.BlockSpec((1, FT, 1024), lambda t, f, txs, tgrp, tval: (tgrp[t], f * tval[t], 0)),
            pl.BlockSpec((1, 1, 1024), lambda t, f, txs, tgrp, tval: (tgrp[t], 0, 0)),
        ],
        out_specs=pl.BlockSpec((TILE, 1024), lambda t, f, txs, tgrp, tval: (txs[t], 0)),
        scratch_shapes=[pltpu.VMEM((TILE, 1024), F32),
                        pltpu.VMEM((TILE, 1), F32),
                        pltpu.VMEM((TILE, 1024), F32)],
    )
    return pl.pallas_call(
        _ffn_body,
        grid_spec=grid_spec,
        out_shape=jax.ShapeDtypeStruct((NSLOTL, 1024), F32),
        interpret=_IT,
    )(txs, tgrp, tval, paR, pbR, gaC, gbC, xn, W1,
      b1.reshape(8 * NF, 1, FT), W2, b2.reshape(8, 1, 1024))


def _combine_body(ys, pk, out, pk_v, pa_v, pb_v, rA_v, rB_v, sem, sem2):
    wid = lax.axis_index("s") * 2 + lax.axis_index("c")
    r0 = wid * 16
    pltpu.sync_copy(pk.at[pl.ds(r0, 16)], pk_v)
    pkv = pk_v[...]
    pa_v[...] = lax.bitwise_and(pkv, 2047)
    pb_v[...] = lax.shift_right_logical(pkv, 11)
    ca = pltpu.async_copy(ys.at[pa_v], rA_v, sem)
    cb = pltpu.async_copy(ys.at[pb_v], rB_v, sem2)
    ca.wait()
    cb.wait()

    def addrow(j, carry):
        for kk in range(64):
            sl = pl.ds(kk * 16, 16)
            rA_v[j, sl] = rA_v[j, sl] + rB_v[j, sl]
        return carry
    lax.fori_loop(0, 16, addrow, 0)
    pltpu.sync_copy(rA_v, out.at[pl.ds(r0, 16)])


def _combine_sc(ys, pk):
    out_type = jax.ShapeDtypeStruct((512, 1024), F32)
    scratch = [
        pltpu.VMEM((16,), jnp.int32),
        pltpu.VMEM((16,), jnp.int32),
        pltpu.VMEM((16,), jnp.int32),
        pltpu.VMEM((16, 1024), F32),
        pltpu.VMEM((16, 1024), F32),
        pltpu.SemaphoreType.DMA,
        pltpu.SemaphoreType.DMA,
    ]
    fn = pl.kernel(_combine_body, out_type=out_type, mesh=_SC_MESH,
                   scratch_types=scratch, compiler_params=_SC_PARAMS)
    return fn(ys, pk)


def _head_body(sec_r, Wc_r, bc_r, feat_o, cls_o):
    rows = [jnp.mean(sec_r[bb * 64:(bb + 1) * 64, :], axis=0, keepdims=True)
            for bb in range(8)]
    feat = jnp.concatenate(rows, axis=0)
    feat_o[...] = feat
    cls_o[...] = jnp.dot(feat, Wc_r[...], preferred_element_type=F32) + bc_r[...]


def _head(second_rows, Wc, bc):
    outs = [jax.ShapeDtypeStruct((8, 1024), F32),
            jax.ShapeDtypeStruct((8, 10), F32)]
    return pl.pallas_call(_head_body, out_shape=outs, interpret=_IT)(
        second_rows, Wc, bc)


def kernel(x, W_patch, b_patch, Wq, Wk, Wv, Wo, bo, pos_emb, ln1_g, ln1_b,
           ln2_g, ln2_b, ln3_g, ln3_b, m1_Wg, m1_bg, m1_W1, m1_b1, m1_W2,
           m1_b2, m2_Wg, m2_bg, m2_W1, m2_b1, m2_W2, m2_b2, Wc, bc):
    b, c, h, w = x.shape
    P = 4
    hp, wp = h // P, w // P
    t = x.reshape(b, c, hp, P, wp, P).transpose(0, 1, 2, 4, 3, 5)
    t = t.reshape(b, c, hp * wp, P * P).transpose(0, 2, 1, 3)
    patches = t.reshape(b * hp * wp, c * P * P)
    pos512 = jnp.tile(pos_emb[0], (b, 1))
    row = lambda a: a.reshape(1, -1)

    (xn2b, xn3b, pk1, gc1, tg1, tx1, tv1,
     pk2, gc2, tg2, tx2, tv2) = _stage_a(
        patches, W_patch, row(b_patch), Wq, Wk, Wv, Wo, row(bo), pos512,
        row(ln1_g), row(ln1_b), row(ln2_g), row(ln2_b), row(ln3_g),
        row(ln3_b), m1_Wg, row(m1_bg), m2_Wg, row(m2_bg))

    pk1 = pk1.reshape(512)
    pk2 = pk2.reshape(512)
    pa1 = (pk1 & 2047).reshape(1, 512)
    pb1 = (pk1 >> 11).reshape(1, 512)
    pa2 = (pk2 & 2047).reshape(1, 512)
    pb2 = (pk2 >> 11).reshape(1, 512)
    ga1, gb1 = gc1[:512], gc1[512:]
    ga2, gb2 = gc2[:512], gc2[512:]

    ys1 = _ffn_grouped(xn2b, pa1, pb1, ga1, gb1, tx1.reshape(MTL),
                       tg1.reshape(MTL), tv1.reshape(MTL),
                       m1_W1, m1_b1, m1_W2, m1_b2)
    ys2 = _ffn_grouped(xn3b, pa2, pb2, ga2, gb2, tx2.reshape(MTL),
                       tg2.reshape(MTL), tv2.reshape(MTL),
                       m2_W1, m2_b1, m2_W2, m2_b2)

    out1 = _combine_sc(ys1, pk1)
    out2 = _combine_sc(ys2, pk2)
    first = out1.reshape(b, 64, 1024)
    second = out2.reshape(b, 64, 1024)
    feat, cls = _head(out2, Wc, row(bc))
    return first, second, feat, cls
